# Initial kernel scaffold; baseline (speedup 1.0000x reference)
import jax
import jax.numpy as jnp
from jax.experimental import pallas as pl


def kernel(x, embeddings):
    raise NotImplementedError("write your pallas kernel here")


# speedup vs baseline: 2.2561x; 2.2561x over previous
import jax
import jax.numpy as jnp
from jax.experimental import pallas as pl

_BETA = 0.25
_K = 1024
_D = 64
_T = 1024


def _vq_body(x_ref, e_ref, q_ref, loss_ref):
    e = e_ref[:]
    xb = x_ref[:]
    sim = jnp.dot(xb, e, preferred_element_type=jnp.float32)
    e2 = jnp.sum(e * e, axis=0, keepdims=True)
    dist = e2 - 2.0 * sim
    idx = jnp.argmin(dist, axis=1)
    onehot = (
        jax.lax.broadcasted_iota(jnp.int32, (_T, _K), 1) == idx[:, None]
    ).astype(jnp.float32)
    q = jax.lax.dot_general(
        onehot, e, (((1,), (1,)), ((), ())),
        preferred_element_type=jnp.float32,
    )
    q_ref[:] = q

    part = jnp.sum(jnp.min(dist, axis=1)) + jnp.sum(xb * xb)

    @pl.when(pl.program_id(0) == 0)
    def _():
        loss_ref[:, :] = jnp.zeros((1, 1), jnp.float32)

    loss_ref[:, :] += jnp.full((1, 1), part)


def kernel(x, embeddings):
    n = x.shape[0] * x.shape[1] * x.shape[2]
    xf = x.reshape(n, _D)
    q, loss_sum = pl.pallas_call(
        _vq_body,
        grid=(n // _T,),
        in_specs=[
            pl.BlockSpec((_T, _D), lambda i: (i, 0)),
            pl.BlockSpec((_D, _K), lambda i: (0, 0)),
        ],
        out_specs=[
            pl.BlockSpec((_T, _D), lambda i: (i, 0)),
            pl.BlockSpec((1, 1), lambda i: (0, 0)),
        ],
        out_shape=[
            jax.ShapeDtypeStruct((n, _D), jnp.float32),
            jax.ShapeDtypeStruct((1, 1), jnp.float32),
        ],
    )(xf, embeddings)
    vq_loss = loss_sum[0, 0] * ((1.0 + _BETA) / (n * _D))
    return q.reshape(x.shape), vq_loss
